# baseline (speedup 1.0000x reference)
import functools

import jax
import jax.numpy as jnp
from jax import lax
from jax.experimental import pallas as pl
from jax.experimental.pallas import tpu as pltpu
from jax.experimental.pallas import tpu_sc as plsc

_EMB = 64
_BATCH = 16384
_B_TOT = 3 * _BATCH
_NC, _NS = 2, 16
_NW = _NC * _NS
_B_PER_W = _B_TOT // _NW
_HALF = _B_PER_W // 2
_NSEM = 4
_REG_W = 1e-5
_N_ROWS = 2_000_000
_TBLK = 16000


def _transpose_body(tt_ref, out_ref):
    out_ref[...] = tt_ref[...].T


def _tc_transpose(table_t):
    return pl.pallas_call(
        _transpose_body,
        grid=(_N_ROWS // _TBLK,),
        in_specs=[pl.BlockSpec((_EMB, _TBLK), lambda i: (0, i))],
        out_specs=pl.BlockSpec((_TBLK, _EMB), lambda i: (i, 0)),
        out_shape=jax.ShapeDtypeStruct((_N_ROWS, _EMB), jnp.float32),
        compiler_params=pltpu.CompilerParams(
            dimension_semantics=("parallel",)
        ),
    )(table_t)


def _sc_gather(table, idx):
    mesh = plsc.VectorSubcoreMesh(core_axis_name="c", subcore_axis_name="s")

    @functools.partial(
        pl.kernel,
        mesh=mesh,
        compiler_params=pltpu.CompilerParams(use_tc_tiling_on_sc=True),
        out_type=jax.ShapeDtypeStruct((_B_TOT, _EMB), jnp.float32),
        scratch_types=[
            pltpu.VMEM((_B_PER_W,), jnp.int32),
            pltpu.VMEM((_HALF, _EMB), jnp.float32),
            pltpu.SemaphoreType.DMA,
            pltpu.SemaphoreType.DMA,
            pltpu.SemaphoreType.DMA,
            pltpu.SemaphoreType.DMA,
            pltpu.SemaphoreType.DMA,
        ],
    )
    def gather_kernel(
        table_hbm, idx_hbm, out_hbm, idx_v, rows_v, sem_i, s0, s1, s2, s3
    ):
        sems = (s0, s1, s2, s3)
        wid = lax.axis_index("s") * _NC + lax.axis_index("c")
        base = wid * _B_PER_W
        pltpu.async_copy(idx_hbm.at[pl.ds(base, _B_PER_W)], idx_v, sem_i).wait()
        for h in range(2):

            @pl.loop(0, _HALF, step=16)
            def _(g):
                vec = idx_v[pl.ds(h * _HALF + g, 16)]
                for j in range(16):
                    pltpu.async_copy(
                        table_hbm.at[pl.ds(vec[j], 1)],
                        rows_v.at[pl.ds(g + j, 1)],
                        sems[j % _NSEM],
                    )

            for k in range(_NSEM):
                pltpu.make_async_copy(
                    table_hbm.at[pl.ds(0, _HALF // _NSEM)],
                    rows_v.at[pl.ds(0, _HALF // _NSEM)],
                    sems[k],
                ).wait()
            pltpu.sync_copy(rows_v, out_hbm.at[pl.ds(base + h * _HALF, _HALF)])

    return gather_kernel(table, idx)


_N_STEP = 8
_ROWS = 128 // _N_STEP


def _tc_body(g_ref, reward_ref, bpr_ref, reg_ref, loss_ref, acc_ref):
    step = pl.program_id(0)

    @pl.when(step == 0)
    def _():
        acc_ref[0] = 0.0
        acc_ref[1] = 0.0

    u = g_ref[0]
    p = g_ref[1]
    n = g_ref[2]
    pos_s = jnp.sum(u * p, axis=2)
    neg_s = jnp.sum(u * n, axis=2)
    ij = jnp.sum(p * n, axis=2)
    reward_ref[...] = neg_s + ij
    x = pos_s - neg_s
    acc_ref[0] += jnp.sum(jnp.log(jax.nn.sigmoid(x)))
    acc_ref[1] += jnp.sum(u * u) + jnp.sum(p * p) + jnp.sum(n * n)

    @pl.when(step == _N_STEP - 1)
    def _():
        bpr = -acc_ref[0] / _BATCH
        reg = _REG_W * 0.5 * acc_ref[1]
        bpr_ref[...] = jnp.full((1, 1), bpr, dtype=jnp.float32)
        reg_ref[...] = jnp.full((1, 1), reg, dtype=jnp.float32)
        loss_ref[...] = jnp.full((1, 1), bpr + reg, dtype=jnp.float32)


def _tc_compute(g4):
    one = jax.ShapeDtypeStruct((1, 1), jnp.float32)
    one_spec = pl.BlockSpec((1, 1), lambda i: (0, 0))
    return pl.pallas_call(
        _tc_body,
        grid=(_N_STEP,),
        in_specs=[
            pl.BlockSpec((3, _ROWS, 128, _EMB), lambda i: (0, i, 0, 0)),
        ],
        out_specs=[
            pl.BlockSpec((_ROWS, 128), lambda i: (i, 0)),
            one_spec,
            one_spec,
            one_spec,
        ],
        out_shape=[
            jax.ShapeDtypeStruct((128, 128), jnp.float32),
            one,
            one,
            one,
        ],
        scratch_shapes=[pltpu.SMEM((2,), jnp.float32)],
    )(g4)


def kernel(all_embed, u_id, pos_i_id, neg_i_id):
    table = _tc_transpose(all_embed.T)
    idx = jnp.concatenate([u_id, pos_i_id, neg_i_id]).astype(jnp.int32)
    g = _sc_gather(table, idx)
    g4 = g.reshape(3, 128, 128, _EMB)
    reward, bpr, reg, loss = _tc_compute(g4)
    return reward.reshape(_BATCH), loss[0, 0], bpr[0, 0], reg[0, 0]


# speedup vs baseline: 1.0752x; 1.0752x over previous
import functools

import jax
import jax.numpy as jnp
from jax import lax
from jax.experimental import pallas as pl
from jax.experimental.pallas import tpu as pltpu
from jax.experimental.pallas import tpu_sc as plsc

_EMB = 64
_PAIR = 128
_BATCH = 16384
_B_TOT = 3 * _BATCH
_NC, _NS = 2, 16
_NW = _NC * _NS
_B_PER_W = _B_TOT // _NW
_HALF = _B_PER_W // 2
_NSEM = 4
_REG_W = 1e-5
_N_ROWS = 2_000_000
_TBLK = 16000


def _transpose_body(tt_ref, out_ref):
    t = tt_ref[...].T
    out_ref[...] = jnp.concatenate(
        [t[0 : _TBLK // 2], t[_TBLK // 2 : _TBLK]], axis=1
    )


def _tc_transpose(table_t):
    return pl.pallas_call(
        _transpose_body,
        grid=(_N_ROWS // _TBLK,),
        in_specs=[pl.BlockSpec((_EMB, _TBLK), lambda i: (0, i))],
        out_specs=pl.BlockSpec((_TBLK // 2, _PAIR), lambda i: (i, 0)),
        out_shape=jax.ShapeDtypeStruct((_N_ROWS // 2, _PAIR), jnp.float32),
        compiler_params=pltpu.CompilerParams(
            dimension_semantics=("arbitrary",)
        ),
    )(table_t)


def _sc_gather(table2, pidx):
    mesh = plsc.VectorSubcoreMesh(core_axis_name="c", subcore_axis_name="s")

    @functools.partial(
        pl.kernel,
        mesh=mesh,
        compiler_params=pltpu.CompilerParams(use_tc_tiling_on_sc=True),
        out_type=jax.ShapeDtypeStruct((_B_TOT, _PAIR), jnp.float32),
        scratch_types=[
            pltpu.VMEM((_B_PER_W,), jnp.int32),
            pltpu.VMEM((_HALF, _PAIR), jnp.float32),
            pltpu.SemaphoreType.DMA,
            pltpu.SemaphoreType.DMA,
            pltpu.SemaphoreType.DMA,
            pltpu.SemaphoreType.DMA,
            pltpu.SemaphoreType.DMA,
        ],
    )
    def gather_kernel(
        table_hbm, idx_hbm, out_hbm, idx_v, rows_v, sem_i, s0, s1, s2, s3
    ):
        sems = (s0, s1, s2, s3)
        wid = lax.axis_index("s") * _NC + lax.axis_index("c")
        base = wid * _B_PER_W
        pltpu.async_copy(idx_hbm.at[pl.ds(base, _B_PER_W)], idx_v, sem_i).wait()
        for h in range(2):

            @pl.loop(0, _HALF, step=16)
            def _(g):
                vec = idx_v[pl.ds(h * _HALF + g, 16)]
                for j in range(16):
                    pltpu.async_copy(
                        table_hbm.at[pl.ds(vec[j], 1)],
                        rows_v.at[pl.ds(g + j, 1)],
                        sems[j % _NSEM],
                    )

            for k in range(_NSEM):
                pltpu.make_async_copy(
                    table_hbm.at[pl.ds(0, _HALF // _NSEM)],
                    rows_v.at[pl.ds(0, _HALF // _NSEM)],
                    sems[k],
                ).wait()
            pltpu.sync_copy(rows_v, out_hbm.at[pl.ds(base + h * _HALF, _HALF)])

    return gather_kernel(table2, pidx)


_N_STEP = 8
_ROWS = 128 // _N_STEP


def _align(rows, par):
    rolled = jnp.concatenate([rows[..., _EMB:], rows[..., :_EMB]], axis=-1)
    sel = jnp.where(par[..., None] > 0.5, rolled, rows)
    lane = lax.broadcasted_iota(jnp.int32, sel.shape, 2)
    return jnp.where(lane < _EMB, sel, 0.0)


def _tc_body(g_ref, par_ref, reward_ref, bpr_ref, reg_ref, loss_ref, acc_ref):
    step = pl.program_id(0)

    @pl.when(step == 0)
    def _():
        acc_ref[0] = 0.0
        acc_ref[1] = 0.0

    par = par_ref[...]
    u = _align(g_ref[0], par[0])
    p = _align(g_ref[1], par[1])
    n = _align(g_ref[2], par[2])
    pos_s = jnp.sum(u * p, axis=2)
    neg_s = jnp.sum(u * n, axis=2)
    ij = jnp.sum(p * n, axis=2)
    reward_ref[...] = neg_s + ij
    x = pos_s - neg_s
    acc_ref[0] += jnp.sum(jnp.log(jax.nn.sigmoid(x)))
    acc_ref[1] += jnp.sum(u * u) + jnp.sum(p * p) + jnp.sum(n * n)

    @pl.when(step == _N_STEP - 1)
    def _():
        bpr = -acc_ref[0] / _BATCH
        reg = _REG_W * 0.5 * acc_ref[1]
        bpr_ref[...] = jnp.full((1, 1), bpr, dtype=jnp.float32)
        reg_ref[...] = jnp.full((1, 1), reg, dtype=jnp.float32)
        loss_ref[...] = jnp.full((1, 1), bpr + reg, dtype=jnp.float32)


def _tc_compute(g4, par3):
    one = jax.ShapeDtypeStruct((1, 1), jnp.float32)
    one_spec = pl.BlockSpec((1, 1), lambda i: (0, 0))
    return pl.pallas_call(
        _tc_body,
        grid=(_N_STEP,),
        in_specs=[
            pl.BlockSpec((3, _ROWS, 128, _PAIR), lambda i: (0, i, 0, 0)),
            pl.BlockSpec((3, _ROWS, 128), lambda i: (0, i, 0)),
        ],
        out_specs=[
            pl.BlockSpec((_ROWS, 128), lambda i: (i, 0)),
            one_spec,
            one_spec,
            one_spec,
        ],
        out_shape=[
            jax.ShapeDtypeStruct((128, 128), jnp.float32),
            one,
            one,
            one,
        ],
        scratch_shapes=[pltpu.SMEM((2,), jnp.float32)],
    )(g4, par3)


def kernel(all_embed, u_id, pos_i_id, neg_i_id):
    table2 = _tc_transpose(all_embed.T)
    idx = jnp.concatenate([u_id, pos_i_id, neg_i_id]).astype(jnp.int32)
    blk = idx // _TBLK
    off = idx % _TBLK
    hi = off >= (_TBLK // 2)
    pidx = blk * (_TBLK // 2) + jnp.where(hi, off - _TBLK // 2, off)
    par = lax.convert_element_type(hi, jnp.float32)
    g = _sc_gather(table2, pidx)
    g4 = g.reshape(3, 128, 128, _PAIR)
    par3 = par.reshape(3, 128, 128)
    reward, bpr, reg, loss = _tc_compute(g4, par3)
    return reward.reshape(_BATCH), loss[0, 0], bpr[0, 0], reg[0, 0]


# speedup vs baseline: 1.0754x; 1.0001x over previous
import functools

import jax
import jax.numpy as jnp
from jax import lax
from jax.experimental import pallas as pl
from jax.experimental.pallas import tpu as pltpu
from jax.experimental.pallas import tpu_sc as plsc

_EMB = 64
_PAIR = 128
_BATCH = 16384
_B_TOT = 3 * _BATCH
_NC, _NS = 2, 16
_NW = _NC * _NS
_B_PER_W = _B_TOT // _NW
_HALF = _B_PER_W // 2
_NSEM = 4
_REG_W = 1e-5
_N_ROWS = 2_000_000
_TBLK = 16000


def _transpose_body(tt_ref, out_ref):
    t = tt_ref[...].T
    out_ref[:, 0:_EMB] = t[0 : _TBLK // 2]
    out_ref[:, _EMB:_PAIR] = t[_TBLK // 2 : _TBLK]


def _tc_transpose(table_t):
    return pl.pallas_call(
        _transpose_body,
        grid=(_N_ROWS // _TBLK,),
        in_specs=[pl.BlockSpec((_EMB, _TBLK), lambda i: (0, i))],
        out_specs=pl.BlockSpec((_TBLK // 2, _PAIR), lambda i: (i, 0)),
        out_shape=jax.ShapeDtypeStruct((_N_ROWS // 2, _PAIR), jnp.float32),
        compiler_params=pltpu.CompilerParams(
            dimension_semantics=("arbitrary",)
        ),
    )(table_t)


def _sc_gather(table2, pidx):
    mesh = plsc.VectorSubcoreMesh(core_axis_name="c", subcore_axis_name="s")

    @functools.partial(
        pl.kernel,
        mesh=mesh,
        compiler_params=pltpu.CompilerParams(use_tc_tiling_on_sc=True),
        out_type=jax.ShapeDtypeStruct((_B_TOT, _PAIR), jnp.float32),
        scratch_types=[
            pltpu.VMEM((_B_PER_W,), jnp.int32),
            pltpu.VMEM((_HALF, _PAIR), jnp.float32),
            pltpu.SemaphoreType.DMA,
            pltpu.SemaphoreType.DMA,
            pltpu.SemaphoreType.DMA,
            pltpu.SemaphoreType.DMA,
            pltpu.SemaphoreType.DMA,
        ],
    )
    def gather_kernel(
        table_hbm, idx_hbm, out_hbm, idx_v, rows_v, sem_i, s0, s1, s2, s3
    ):
        sems = (s0, s1, s2, s3)
        wid = lax.axis_index("s") * _NC + lax.axis_index("c")
        base = wid * _B_PER_W
        pltpu.async_copy(idx_hbm.at[pl.ds(base, _B_PER_W)], idx_v, sem_i).wait()
        for h in range(2):

            @pl.loop(0, _HALF, step=16)
            def _(g):
                vec = idx_v[pl.ds(h * _HALF + g, 16)]
                for j in range(16):
                    pltpu.async_copy(
                        table_hbm.at[pl.ds(vec[j], 1)],
                        rows_v.at[pl.ds(g + j, 1)],
                        sems[j % _NSEM],
                    )

            for k in range(_NSEM):
                pltpu.make_async_copy(
                    table_hbm.at[pl.ds(0, _HALF // _NSEM)],
                    rows_v.at[pl.ds(0, _HALF // _NSEM)],
                    sems[k],
                ).wait()
            pltpu.sync_copy(rows_v, out_hbm.at[pl.ds(base + h * _HALF, _HALF)])

    return gather_kernel(table2, pidx)


_N_STEP = 8
_ROWS = 128 // _N_STEP


def _align(rows, par):
    rolled = jnp.concatenate([rows[..., _EMB:], rows[..., :_EMB]], axis=-1)
    sel = jnp.where(par[..., None] > 0.5, rolled, rows)
    lane = lax.broadcasted_iota(jnp.int32, sel.shape, 2)
    return jnp.where(lane < _EMB, sel, 0.0)


def _tc_body(g_ref, par_ref, reward_ref, bpr_ref, reg_ref, loss_ref, acc_ref):
    step = pl.program_id(0)

    @pl.when(step == 0)
    def _():
        acc_ref[0] = 0.0
        acc_ref[1] = 0.0

    par = par_ref[...]
    u = _align(g_ref[0], par[0])
    p = _align(g_ref[1], par[1])
    n = _align(g_ref[2], par[2])
    pos_s = jnp.sum(u * p, axis=2)
    neg_s = jnp.sum(u * n, axis=2)
    ij = jnp.sum(p * n, axis=2)
    reward_ref[...] = neg_s + ij
    x = pos_s - neg_s
    acc_ref[0] += jnp.sum(jnp.log(jax.nn.sigmoid(x)))
    acc_ref[1] += jnp.sum(u * u) + jnp.sum(p * p) + jnp.sum(n * n)

    @pl.when(step == _N_STEP - 1)
    def _():
        bpr = -acc_ref[0] / _BATCH
        reg = _REG_W * 0.5 * acc_ref[1]
        bpr_ref[...] = jnp.full((1, 1), bpr, dtype=jnp.float32)
        reg_ref[...] = jnp.full((1, 1), reg, dtype=jnp.float32)
        loss_ref[...] = jnp.full((1, 1), bpr + reg, dtype=jnp.float32)


def _tc_compute(g4, par3):
    one = jax.ShapeDtypeStruct((1, 1), jnp.float32)
    one_spec = pl.BlockSpec((1, 1), lambda i: (0, 0))
    return pl.pallas_call(
        _tc_body,
        grid=(_N_STEP,),
        in_specs=[
            pl.BlockSpec((3, _ROWS, 128, _PAIR), lambda i: (0, i, 0, 0)),
            pl.BlockSpec((3, _ROWS, 128), lambda i: (0, i, 0)),
        ],
        out_specs=[
            pl.BlockSpec((_ROWS, 128), lambda i: (i, 0)),
            one_spec,
            one_spec,
            one_spec,
        ],
        out_shape=[
            jax.ShapeDtypeStruct((128, 128), jnp.float32),
            one,
            one,
            one,
        ],
        scratch_shapes=[pltpu.SMEM((2,), jnp.float32)],
    )(g4, par3)


def kernel(all_embed, u_id, pos_i_id, neg_i_id):
    table2 = _tc_transpose(all_embed.T)
    idx = jnp.concatenate([u_id, pos_i_id, neg_i_id]).astype(jnp.int32)
    blk = idx // _TBLK
    off = idx % _TBLK
    hi = off >= (_TBLK // 2)
    pidx = blk * (_TBLK // 2) + jnp.where(hi, off - _TBLK // 2, off)
    par = lax.convert_element_type(hi, jnp.float32)
    g = _sc_gather(table2, pidx)
    g4 = g.reshape(3, 128, 128, _PAIR)
    par3 = par.reshape(3, 128, 128)
    reward, bpr, reg, loss = _tc_compute(g4, par3)
    return reward.reshape(_BATCH), loss[0, 0], bpr[0, 0], reg[0, 0]
